# Initial kernel scaffold; baseline (speedup 1.0000x reference)
import jax
import jax.numpy as jnp
from jax.experimental import pallas as pl


def kernel(x, edge_index, W_l, W_r, att, bias):
    raise NotImplementedError("write your pallas kernel here")


# speedup vs baseline: 5.6131x; 5.6131x over previous
import functools

import jax
import jax.numpy as jnp
from jax import lax
from jax.experimental import pallas as pl
from jax.experimental.pallas import tpu as pltpu
from jax.experimental.pallas import tpu_sc as plsc

NC = 2
NS = 16
NW = NC * NS
C = 80


def _matmuls(x, W_l, W_r):
    n, d = x.shape
    mblk = 400
    body = lambda x_ref, wl_ref, wr_ref, xl_ref, xr_ref: (
        xl_ref.__setitem__(..., jnp.dot(x_ref[...], wl_ref[...],
                                        preferred_element_type=jnp.float32)),
        xr_ref.__setitem__(..., jnp.dot(x_ref[...], wr_ref[...],
                                        preferred_element_type=jnp.float32)),
    ) and None
    return pl.pallas_call(
        body,
        grid=(n // mblk,),
        in_specs=[
            pl.BlockSpec((mblk, d), lambda i: (i, 0)),
            pl.BlockSpec((d, d), lambda i: (0, 0)),
            pl.BlockSpec((d, d), lambda i: (0, 0)),
        ],
        out_specs=[pl.BlockSpec((mblk, d), lambda i: (i, 0))] * 2,
        out_shape=[jax.ShapeDtypeStruct((n, d), jnp.float32)] * 2,
    )(x, W_l, W_r)


def _edge_logits(xl, xr, src, dst, att):
    n, d = xl.shape
    e_total = src.shape[0]
    ew = e_total // NW
    nchunk = ew // C
    mesh = plsc.VectorSubcoreMesh(core_axis_name="c", subcore_axis_name="s")

    @functools.partial(
        pl.kernel,
        out_type=[
            jax.ShapeDtypeStruct((e_total,), jnp.float32),
            jax.ShapeDtypeStruct((NW, 16), jnp.float32),
        ],
        mesh=mesh,
        compiler_params=pltpu.CompilerParams(needs_layout_passes=False),
        scratch_types=[
            pltpu.VMEM((C,), jnp.int32),
            pltpu.VMEM((C,), jnp.int32),
            pltpu.VMEM((C, d), jnp.float32),
            pltpu.VMEM((C, d), jnp.float32),
            pltpu.VMEM((C,), jnp.float32),
            pltpu.VMEM((d,), jnp.float32),
            pltpu.VMEM((16,), jnp.float32),
            pltpu.VMEM((C * 16,), jnp.float32),
            pltpu.SemaphoreType.DMA,
        ],
    )
    def body(xl_hbm, xr_hbm, src_hbm, dst_hbm, att_hbm, e_hbm, mx_hbm,
             src_v, dst_v, rl_v, rr_v, e_v, att_v, mx_v, acc_v, sem):
        c = lax.axis_index("c")
        s = lax.axis_index("s")
        wid = c * NS + s
        base = wid * ew
        pltpu.sync_copy(att_hbm, att_v)
        lanes0 = lax.iota(jnp.int32, 16)
        att_regs = [att_v[pl.ds(k * 16, 16)] for k in range(d // 16)]

        def chunk_body(j, macc):
            eb = base + j * C
            pltpu.sync_copy(src_hbm.at[pl.ds(eb, C)], src_v)
            pltpu.sync_copy(dst_hbm.at[pl.ds(eb, C)], dst_v)
            pltpu.async_copy(xl_hbm.at[src_v], rl_v, sem).wait()
            pltpu.async_copy(xr_hbm.at[dst_v], rr_v, sem).wait()

            def edge_body(i, _):
                acc = jnp.zeros((16,), jnp.float32)
                for k in range(d // 16):
                    z = rl_v[i, pl.ds(k * 16, 16)] + rr_v[i, pl.ds(k * 16, 16)]
                    z = jnp.maximum(z, 0.2 * z)
                    acc = acc + z * att_regs[k]
                acc_v[pl.ds(i * 16, 16)] = acc
                return 0

            lax.fori_loop(0, C, edge_body, 0)

            def grp_body(g, macc2):
                fbase = g * 256 + lanes0 * 16
                esum = jnp.zeros((16,), jnp.float32)
                for t in range(16):
                    esum = esum + plsc.load_gather(acc_v, [fbase + t])
                e_v[pl.ds(g * 16, 16)] = esum
                return jnp.maximum(macc2, esum)

            macc = lax.fori_loop(0, C // 16, grp_body, macc)
            pltpu.sync_copy(e_v, e_hbm.at[pl.ds(eb, C)])
            return macc

        macc = lax.fori_loop(0, nchunk, chunk_body,
                             jnp.full((16,), -3.4e38, jnp.float32))
        mx_v[...] = macc
        pltpu.sync_copy(mx_v, mx_hbm.at[wid])

    return body(xl, xr, src, dst, att)


def _aggregate(xl, src, dst, e, mx, n_pad):
    n, d = xl.shape
    e_total = src.shape[0]
    ew = e_total // NW
    es = e_total // NS
    rows_per_tile = n_pad // NS
    mesh = plsc.VectorSubcoreMesh(core_axis_name="c", subcore_axis_name="s")

    @functools.partial(
        pl.kernel,
        out_type=jax.ShapeDtypeStruct((NC, n_pad, d), jnp.float32),
        mesh=mesh,
        compiler_params=pltpu.CompilerParams(needs_layout_passes=False),
        scratch_types=[
            pltpu.VMEM_SHARED((n_pad,), jnp.float32),
            pltpu.VMEM_SHARED((n_pad, d), jnp.float32),
            pltpu.VMEM((C,), jnp.int32),
            pltpu.VMEM((C,), jnp.int32),
            pltpu.VMEM((C,), jnp.float32),
            pltpu.VMEM((C, d), jnp.float32),
            pltpu.VMEM((NW, 16), jnp.float32),
            pltpu.VMEM((n_pad,), jnp.float32),
            pltpu.VMEM((16,), jnp.float32),
            pltpu.SemaphoreType.DMA,
        ],
    )
    def body(xl_hbm, src_hbm, dst_hbm, e_hbm, mx_hbm, outp_hbm,
             den_sh, out_sh, src_v, dst_v, e_v, rows_v, mx_v, den_v,
             red_v, sem):
        c = lax.axis_index("c")
        s = lax.axis_index("s")
        wid = c * NS + s
        zeros16 = jnp.zeros((16,), jnp.float32)
        lanes0 = lax.iota(jnp.int32, 16)

        def zrow(i, _):
            for k in range(d // 16):
                rows_v[i, pl.ds(k * 16, 16)] = zeros16
            return 0
        lax.fori_loop(0, C, zrow, 0)

        def zden(i, _):
            den_v[pl.ds(i * 16, 16)] = zeros16
            return 0
        lax.fori_loop(0, n_pad // 16, zden, 0)

        @pl.when(s == 0)
        def _():
            pltpu.sync_copy(den_v, den_sh)

        for t in range(rows_per_tile // C):
            pltpu.sync_copy(
                rows_v, out_sh.at[pl.ds(s * rows_per_tile + t * C, C)])
        plsc.subcore_barrier()

        pltpu.sync_copy(mx_hbm, mx_v)

        def mred(i, mv):
            return jnp.maximum(mv, mx_v[i, :])
        mv = lax.fori_loop(0, NW, mred, jnp.full((16,), -3.4e38, jnp.float32))
        for stp in (1, 2, 4, 8):
            red_v[...] = mv
            mv = jnp.maximum(mv, plsc.load_gather(red_v, [lanes0 ^ stp]))
        M = mv

        def dchunk(j, _):
            eb = s * es + j * C
            pltpu.sync_copy(dst_hbm.at[pl.ds(eb, C)], dst_v)
            pltpu.sync_copy(e_hbm.at[pl.ds(eb, C)], e_v)
            for g in range(C // 16):
                e_v[pl.ds(g * 16, 16)] = jnp.exp(e_v[pl.ds(g * 16, 16)] - M)
            pltpu.sync_copy(e_v, den_sh.at[dst_v], add=True)
            return 0
        lax.fori_loop(0, es // C, dchunk, 0)
        plsc.subcore_barrier()

        pltpu.sync_copy(den_sh, den_v)

        def achunk(j, _):
            eb = wid * ew + j * C
            pltpu.sync_copy(src_hbm.at[pl.ds(eb, C)], src_v)
            pltpu.sync_copy(dst_hbm.at[pl.ds(eb, C)], dst_v)
            pltpu.sync_copy(e_hbm.at[pl.ds(eb, C)], e_v)
            pltpu.async_copy(xl_hbm.at[src_v], rows_v, sem).wait()
            for g in range(C // 16):
                dv = dst_v[pl.ds(g * 16, 16)]
                den = plsc.load_gather(den_v, [dv])
                wv = (jnp.exp(e_v[pl.ds(g * 16, 16)] - M)
                      / jnp.maximum(den, 1e-16))
                e_v[pl.ds(g * 16, 16)] = wv

            def scale_body(i, _2):
                ws = plsc.load_gather(e_v, [jnp.full((16,), i, jnp.int32)])
                for k in range(d // 16):
                    rows_v[i, pl.ds(k * 16, 16)] = (
                        rows_v[i, pl.ds(k * 16, 16)] * ws)
                return 0
            lax.fori_loop(0, C, scale_body, 0)
            pltpu.sync_copy(rows_v, out_sh.at[dst_v], add=True)
            return 0
        lax.fori_loop(0, ew // C, achunk, 0)
        plsc.subcore_barrier()

        pltpu.sync_copy(
            out_sh.at[pl.ds(s * rows_per_tile, rows_per_tile)],
            outp_hbm.at[c, pl.ds(s * rows_per_tile, rows_per_tile)])

    return body(xl, src, dst, e, mx)


def _finalize(a, b, bias2d):
    n_pad, d = a.shape
    fblk = 512

    def body(a_ref, b_ref, bias_ref, o_ref):
        sm = a_ref[...] + b_ref[...] + bias_ref[...]
        neg = jnp.exp(jnp.minimum(sm, 0.0)) - 1.0
        sm = jnp.where(sm > 0, sm, neg)
        z = sm - jnp.max(sm, axis=-1, keepdims=True)
        lse = jnp.log(jnp.sum(jnp.exp(z), axis=-1, keepdims=True))
        o_ref[...] = z - lse

    return pl.pallas_call(
        body,
        grid=(n_pad // fblk,),
        in_specs=[
            pl.BlockSpec((fblk, d), lambda i: (i, 0)),
            pl.BlockSpec((fblk, d), lambda i: (i, 0)),
            pl.BlockSpec((1, d), lambda i: (0, 0)),
        ],
        out_specs=pl.BlockSpec((fblk, d), lambda i: (i, 0)),
        out_shape=jax.ShapeDtypeStruct((n_pad, d), jnp.float32),
    )(a, b, bias2d)


def kernel(x, edge_index, W_l, W_r, att, bias):
    n, d = x.shape
    n_pad = ((n + 511) // 512) * 512
    src = edge_index[0]
    dst = edge_index[1]
    xl, xr = _matmuls(x, W_l, W_r)
    e, mx = _edge_logits(xl, xr, src, dst, att)
    outp = _aggregate(xl, src, dst, e, mx, n_pad)
    res = _finalize(outp[0], outp[1], bias.reshape(1, d))
    return res[:n]


# speedup vs baseline: 14.9072x; 2.6558x over previous
import functools

import jax
import jax.numpy as jnp
from jax import lax
from jax.experimental import pallas as pl
from jax.experimental.pallas import tpu as pltpu
from jax.experimental.pallas import tpu_sc as plsc

NC = 2
NS = 16
NW = NC * NS
C = 80
H0 = 64


def _matmuls(x, W_l, W_r):
    n, d = x.shape
    mblk = 400
    body = lambda x_ref, wl_ref, wr_ref, xl_ref, xr_ref: (
        xl_ref.__setitem__(..., jnp.dot(x_ref[...], wl_ref[...],
                                        preferred_element_type=jnp.float32)),
        xr_ref.__setitem__(..., jnp.dot(x_ref[...], wr_ref[...],
                                        preferred_element_type=jnp.float32)),
    ) and None
    return pl.pallas_call(
        body,
        grid=(n // mblk,),
        in_specs=[
            pl.BlockSpec((mblk, d), lambda i: (i, 0)),
            pl.BlockSpec((d, d), lambda i: (0, 0)),
            pl.BlockSpec((d, d), lambda i: (0, 0)),
        ],
        out_specs=[pl.BlockSpec((mblk, d), lambda i: (i, 0))] * 2,
        out_shape=[jax.ShapeDtypeStruct((n, d), jnp.float32)] * 2,
    )(x, W_l, W_r)


def _edge_logits(xl, xr, src, dst, att):
    n, d = xl.shape
    e_total = src.shape[0]
    ew = e_total // NW
    nchunk = ew // C
    halves = ((0, 48), (48, 48), (96, nchunk - 96))
    hmax = 48
    mesh = plsc.VectorSubcoreMesh(core_axis_name="c", subcore_axis_name="s")

    @functools.partial(
        pl.kernel,
        out_type=[
            jax.ShapeDtypeStruct((e_total,), jnp.float32),
            jax.ShapeDtypeStruct((NW, 16), jnp.float32),
        ],
        mesh=mesh,
        compiler_params=pltpu.CompilerParams(needs_layout_passes=False),
        scratch_types=[
            pltpu.VMEM((hmax * C,), jnp.int32),
            pltpu.VMEM((hmax * C,), jnp.int32),
            pltpu.VMEM((C, d), jnp.float32),
            pltpu.VMEM((C, d), jnp.float32),
            pltpu.VMEM((C, d), jnp.float32),
            pltpu.VMEM((C, d), jnp.float32),
            pltpu.VMEM((hmax * C,), jnp.float32),
            pltpu.VMEM((d,), jnp.float32),
            pltpu.VMEM((16,), jnp.float32),
            pltpu.VMEM((C * 16,), jnp.float32),
            pltpu.SemaphoreType.DMA,
            pltpu.SemaphoreType.DMA,
        ],
    )
    def body(xl_hbm, xr_hbm, src_hbm, dst_hbm, att_hbm, e_hbm, mx_hbm,
             srcb_v, dstb_v, rl0, rr0, rl1, rr1, e_all, att_v, mx_v, acc_v,
             gsem0, gsem1):
        c = lax.axis_index("c")
        s = lax.axis_index("s")
        wid = c * NS + s
        base = wid * ew
        pltpu.sync_copy(att_hbm, att_v)
        lanes0 = lax.iota(jnp.int32, 16)
        lanes16 = lanes0 * 16
        att_regs = [att_v[pl.ds(k * 16, 16)] for k in range(d // 16)]
        bufs = ((rl0, rr0, gsem0), (rl1, rr1, gsem1))

        def issue(j, b):
            rl, rr, sem = bufs[b]
            pltpu.async_copy(xl_hbm.at[srcb_v.at[pl.ds(j * C, C)]], rl, sem)
            pltpu.async_copy(xr_hbm.at[dstb_v.at[pl.ds(j * C, C)]], rr, sem)

        def wait(j, b):
            rl, rr, sem = bufs[b]
            pltpu.make_async_copy(
                xl_hbm.at[srcb_v.at[pl.ds(j * C, C)]], rl, sem).wait()
            pltpu.make_async_copy(
                xr_hbm.at[dstb_v.at[pl.ds(j * C, C)]], rr, sem).wait()

        def compute(j, b, macc):
            rl, rr, _ = bufs[b]

            @plsc.parallel_loop(0, C, unroll=2)
            def edge_body(i):
                acc = jnp.zeros((16,), jnp.float32)
                for k in range(d // 16):
                    z = rl[i, pl.ds(k * 16, 16)] + rr[i, pl.ds(k * 16, 16)]
                    z = jnp.maximum(z, 0.2 * z)
                    acc = acc + z * att_regs[k]
                acc_v[pl.ds(i * 16, 16)] = acc

            def grp_body(g, macc2):
                fbase = g * 256 + lanes16
                esum = jnp.zeros((16,), jnp.float32)
                for t in range(16):
                    esum = esum + plsc.load_gather(acc_v, [fbase + t])
                e_all[pl.ds(j * C + g * 16, 16)] = esum
                return jnp.maximum(macc2, esum)

            return lax.fori_loop(0, C // 16, grp_body, macc)

        macc = jnp.full((16,), -3.4e38, jnp.float32)
        for h_off, cnt in halves:
            hbase = base + h_off * C
            pltpu.sync_copy(src_hbm.at[pl.ds(hbase, cnt * C)],
                            srcb_v.at[pl.ds(0, cnt * C)])
            pltpu.sync_copy(dst_hbm.at[pl.ds(hbase, cnt * C)],
                            dstb_v.at[pl.ds(0, cnt * C)])
            issue(0, 0)
            p = (cnt - 1) // 2

            def pair(t, m2):
                j = t * 2
                issue(j + 1, 1)
                wait(j, 0)
                m2 = compute(j, 0, m2)
                issue(j + 2, 0)
                wait(j + 1, 1)
                return compute(j + 1, 1, m2)

            macc = lax.fori_loop(0, p, pair, macc)
            if cnt % 2 == 1:
                wait(2 * p, 0)
                macc = compute(2 * p, 0, macc)
            else:
                wait(2 * p, 0)
                issue(2 * p + 1, 1)
                macc = compute(2 * p, 0, macc)
                wait(2 * p + 1, 1)
                macc = compute(2 * p + 1, 1, macc)
            pltpu.sync_copy(e_all.at[pl.ds(0, cnt * C)],
                            e_hbm.at[pl.ds(hbase, cnt * C)])

        mx_v[...] = macc
        pltpu.sync_copy(mx_v, mx_hbm.at[wid])

    return body(xl, xr, src, dst, att)


def _aggregate(xl, src, dstR, dstD, e, mx, n_pad):
    n, d = xl.shape
    e_total = src.shape[0]
    ew = e_total // NW
    nag = ew // C
    halves = ((0, H0), (H0, nag - H0))
    es = e_total // NS
    nd = es // C
    nblk = 25
    ndb = nd // nblk
    rpt = n_pad // NS
    dpt = n_pad // 8
    mesh = plsc.VectorSubcoreMesh(core_axis_name="c", subcore_axis_name="s")

    @functools.partial(
        pl.kernel,
        out_type=[
            jax.ShapeDtypeStruct((NC, n_pad, d), jnp.float32),
            jax.ShapeDtypeStruct((n_pad,), jnp.float32),
        ],
        mesh=mesh,
        compiler_params=pltpu.CompilerParams(needs_layout_passes=False),
        scratch_types=[
            pltpu.VMEM_SHARED((n_pad,), jnp.float32),
            pltpu.VMEM_SHARED((n_pad, d), jnp.float32),
            pltpu.VMEM((H0 * C,), jnp.int32),
            pltpu.VMEM((H0, C), jnp.int32),
            pltpu.VMEM((H0 * C,), jnp.float32),
            pltpu.VMEM((ndb, C), jnp.int32),
            pltpu.VMEM((ndb * C,), jnp.float32),
            pltpu.VMEM((C, d), jnp.float32),
            pltpu.VMEM((C, d), jnp.float32),
            pltpu.VMEM((C,), jnp.float32),
            pltpu.VMEM((C,), jnp.float32),
            pltpu.VMEM((NW, 16), jnp.float32),
            pltpu.VMEM((dpt,), jnp.float32),
            pltpu.VMEM((16,), jnp.float32),
            pltpu.SemaphoreType.DMA,
            pltpu.SemaphoreType.DMA,
            pltpu.SemaphoreType.DMA,
        ],
    )
    def body(xl_hbm, src_hbm, dstR_hbm, dstD_hbm, e_hbm, mx_hbm,
             outp_hbm, den_hbm,
             den_sh, out_sh, srcb_v, dstb2, eb2, dstd2, ed2, r0, r1,
             dc0, dc1, mx_v, zbuf, red_v, gs0, gs1, dsem):
        c = lax.axis_index("c")
        s = lax.axis_index("s")
        wid = c * NS + s
        zeros16 = jnp.zeros((16,), jnp.float32)
        lanes0 = lax.iota(jnp.int32, 16)
        pltpu.sync_copy(mx_hbm, mx_v)

        @plsc.parallel_loop(0, C, unroll=2)
        def zrow(i):
            for k in range(d // 16):
                r0[i, pl.ds(k * 16, 16)] = zeros16

        @plsc.parallel_loop(0, dpt // 16, unroll=2)
        def zden(i):
            zbuf[pl.ds(i * 16, 16)] = zeros16

        @pl.when(s < 8)
        def _():
            pltpu.sync_copy(zbuf, den_sh.at[pl.ds(s * dpt, dpt)])
        for t in range(rpt // C):
            pltpu.sync_copy(r0, out_sh.at[pl.ds(s * rpt + t * C, C)])
        if rpt % C:
            pltpu.sync_copy(
                r0.at[pl.ds(0, rpt % C)],
                out_sh.at[pl.ds(s * rpt + (rpt // C) * C, rpt % C)])
        plsc.subcore_barrier()

        def mred(i, mv):
            return jnp.maximum(mv, mx_v[i, :])
        mv = lax.fori_loop(0, NW, mred, jnp.full((16,), -3.4e38, jnp.float32))
        for stp in (1, 2, 4, 8):
            red_v[...] = mv
            mv = jnp.maximum(mv, plsc.load_gather(red_v, [lanes0 ^ stp]))
        M = mv

        def dblock(bk, _):
            pltpu.sync_copy(dstD_hbm.at[s, bk], dstd2)
            pltpu.sync_copy(e_hbm.at[pl.ds(s * es + bk * ndb * C, ndb * C)],
                            ed2)

            def dexp(q, _2):
                ed2[pl.ds(q * 16, 16)] = jnp.exp(ed2[pl.ds(q * 16, 16)] - M)
                return 0
            lax.fori_loop(0, ndb * C // 16, dexp, 0)

            def dfire(q, _2):
                pltpu.async_copy(ed2.at[pl.ds(q * C, C)],
                                 den_sh.at[dstd2.at[q]], dsem, add=True)
                return 0
            lax.fori_loop(0, ndb, dfire, 0)
            pltpu.make_async_copy(
                e_hbm.at[pl.ds(s * es, ndb * C)], ed2, dsem).wait()
            return 0
        lax.fori_loop(0, nblk, dblock, 0)
        plsc.subcore_barrier()

        @pl.when(s < 8)
        def _():
            pltpu.sync_copy(den_sh.at[pl.ds(s * dpt, dpt)], zbuf)
            pltpu.sync_copy(zbuf, den_hbm.at[pl.ds(s * dpt, dpt)])
        plsc.subcore_barrier()

        bufs = ((r0, dc0, gs0), (r1, dc1, gs1))

        def gissue(j, b):
            r, dc, sem = bufs[b]
            pltpu.async_copy(xl_hbm.at[srcb_v.at[pl.ds(j * C, C)]], r, sem)
            pltpu.async_copy(den_hbm.at[dstb2.at[j]], dc, sem)

        def gwait(j, b):
            r, dc, sem = bufs[b]
            pltpu.make_async_copy(
                xl_hbm.at[srcb_v.at[pl.ds(j * C, C)]], r, sem).wait()
            pltpu.make_async_copy(den_hbm.at[dstb2.at[j]], dc, sem).wait()

        def ssync(j, b):
            pltpu.sync_copy(bufs[b][0], out_sh.at[dstb2.at[j]], add=True)

        def scale(j, b):
            r, dc, _ = bufs[b]

            def grp(g, _2):
                off = j * C + g * 16
                den = dc[pl.ds(g * 16, 16)]
                wv = (jnp.exp(eb2[pl.ds(off, 16)] - M)
                      / jnp.maximum(den, 1e-16))
                eb2[pl.ds(off, 16)] = wv
                return 0
            lax.fori_loop(0, C // 16, grp, 0)

            @plsc.parallel_loop(0, C, unroll=2)
            def sc(i):
                ws = plsc.load_gather(eb2, [jnp.full((16,), j * C + i,
                                                     jnp.int32)])
                for k in range(d // 16):
                    r[i, pl.ds(k * 16, 16)] = r[i, pl.ds(k * 16, 16)] * ws

        for h_off, cnt in halves:
            habs = wid * ew + h_off * C
            pltpu.sync_copy(src_hbm.at[pl.ds(habs, cnt * C)],
                            srcb_v.at[pl.ds(0, cnt * C)])
            pltpu.sync_copy(dstR_hbm.at[wid, pl.ds(h_off, cnt)],
                            dstb2.at[pl.ds(0, cnt)])
            pltpu.sync_copy(e_hbm.at[pl.ds(habs, cnt * C)],
                            eb2.at[pl.ds(0, cnt * C)])
            gissue(0, 0)
            p = (cnt - 1) // 2

            def pair(t, _):
                j = t * 2
                gwait(j, 0)
                gissue(j + 1, 1)
                scale(j, 0)
                ssync(j, 0)
                gwait(j + 1, 1)
                gissue(j + 2, 0)
                scale(j + 1, 1)
                ssync(j + 1, 1)
                return 0
            lax.fori_loop(0, p, pair, 0)
            if cnt % 2 == 1:
                gwait(2 * p, 0)
                scale(2 * p, 0)
                ssync(2 * p, 0)
            else:
                gwait(2 * p, 0)
                gissue(2 * p + 1, 1)
                scale(2 * p, 0)
                ssync(2 * p, 0)
                gwait(2 * p + 1, 1)
                scale(2 * p + 1, 1)
                ssync(2 * p + 1, 1)
        plsc.subcore_barrier()

        pltpu.sync_copy(out_sh.at[pl.ds(s * rpt, rpt)],
                        outp_hbm.at[c, pl.ds(s * rpt, rpt)])

    return body(xl, src, dstR, dstD, e, mx)


def _finalize(a, b, bias2d):
    n_pad, d = a.shape
    fblk = 128

    def body(a_ref, b_ref, bias_ref, o_ref):
        sm = a_ref[...] + b_ref[...] + bias_ref[...]
        neg = jnp.exp(jnp.minimum(sm, 0.0)) - 1.0
        sm = jnp.where(sm > 0, sm, neg)
        z = sm - jnp.max(sm, axis=-1, keepdims=True)
        lse = jnp.log(jnp.sum(jnp.exp(z), axis=-1, keepdims=True))
        o_ref[...] = z - lse

    return pl.pallas_call(
        body,
        grid=(n_pad // fblk,),
        in_specs=[
            pl.BlockSpec((fblk, d), lambda i: (i, 0)),
            pl.BlockSpec((fblk, d), lambda i: (i, 0)),
            pl.BlockSpec((1, d), lambda i: (0, 0)),
        ],
        out_specs=pl.BlockSpec((fblk, d), lambda i: (i, 0)),
        out_shape=jax.ShapeDtypeStruct((n_pad, d), jnp.float32),
    )(a, b, bias2d)


def kernel(x, edge_index, W_l, W_r, att, bias):
    n, d = x.shape
    n_pad = ((n + 127) // 128) * 128
    e_total = edge_index.shape[1]
    ew = e_total // NW
    es = e_total // NS
    src = edge_index[0]
    dst = edge_index[1]
    dstR = dst.reshape(NW, ew // C, C)
    dstD = dst.reshape(NS, 25, es // C // 25, C)
    xl, xr = _matmuls(x, W_l, W_r)
    e, mx = _edge_logits(xl, xr, src, dst, att)
    outp, _den = _aggregate(xl, src, dstR, dstD, e, mx, n_pad)
    res = _finalize(outp[0], outp[1], bias.reshape(1, d))
    return res[:n]


# speedup vs baseline: 15.9024x; 1.0668x over previous
import functools

import jax
import jax.numpy as jnp
from jax import lax
from jax.experimental import pallas as pl
from jax.experimental.pallas import tpu as pltpu
from jax.experimental.pallas import tpu_sc as plsc

NC = 2
NS = 16
NW = NC * NS
C = 80
H0 = 64


def _matmuls(x, W_l, W_r):
    n, d = x.shape
    mblk = 400
    body = lambda x_ref, wl_ref, wr_ref, xl_ref, xr_ref: (
        xl_ref.__setitem__(..., jnp.dot(x_ref[...], wl_ref[...],
                                        preferred_element_type=jnp.float32)),
        xr_ref.__setitem__(..., jnp.dot(x_ref[...], wr_ref[...],
                                        preferred_element_type=jnp.float32)),
    ) and None
    return pl.pallas_call(
        body,
        grid=(n // mblk,),
        in_specs=[
            pl.BlockSpec((mblk, d), lambda i: (i, 0)),
            pl.BlockSpec((d, d), lambda i: (0, 0)),
            pl.BlockSpec((d, d), lambda i: (0, 0)),
        ],
        out_specs=[pl.BlockSpec((mblk, d), lambda i: (i, 0))] * 2,
        out_shape=[jax.ShapeDtypeStruct((n, d), jnp.float32)] * 2,
    )(x, W_l, W_r)


def _edge_logits(xl, xr, src, dst, att):
    n, d = xl.shape
    e_total = src.shape[0]
    ew = e_total // NW
    nchunk = ew // C
    halves = ((0, 48), (48, 48), (96, nchunk - 96))
    hmax = 48
    mesh = plsc.VectorSubcoreMesh(core_axis_name="c", subcore_axis_name="s")

    @functools.partial(
        pl.kernel,
        out_type=[
            jax.ShapeDtypeStruct((e_total,), jnp.float32),
            jax.ShapeDtypeStruct((NW, 16), jnp.float32),
        ],
        mesh=mesh,
        compiler_params=pltpu.CompilerParams(needs_layout_passes=False),
        scratch_types=[
            pltpu.VMEM((hmax * C,), jnp.int32),
            pltpu.VMEM((hmax * C,), jnp.int32),
            pltpu.VMEM((C, d), jnp.float32),
            pltpu.VMEM((C, d), jnp.float32),
            pltpu.VMEM((C, d), jnp.float32),
            pltpu.VMEM((C, d), jnp.float32),
            pltpu.VMEM((hmax * C,), jnp.float32),
            pltpu.VMEM((d,), jnp.float32),
            pltpu.VMEM((16,), jnp.float32),
            pltpu.VMEM((C * 16,), jnp.float32),
            pltpu.SemaphoreType.DMA,
            pltpu.SemaphoreType.DMA,
        ],
    )
    def body(xl_hbm, xr_hbm, src_hbm, dst_hbm, att_hbm, e_hbm, mx_hbm,
             srcb_v, dstb_v, rl0, rr0, rl1, rr1, e_all, att_v, mx_v, acc_v,
             gsem0, gsem1):
        c = lax.axis_index("c")
        s = lax.axis_index("s")
        wid = c * NS + s
        base = wid * ew
        pltpu.sync_copy(att_hbm, att_v)
        lanes0 = lax.iota(jnp.int32, 16)
        lanes16 = lanes0 * 16
        att_regs = [att_v[pl.ds(k * 16, 16)] for k in range(d // 16)]
        bufs = ((rl0, rr0, gsem0), (rl1, rr1, gsem1))

        def issue(j, b):
            rl, rr, sem = bufs[b]
            pltpu.async_copy(xl_hbm.at[srcb_v.at[pl.ds(j * C, C)]], rl, sem)
            pltpu.async_copy(xr_hbm.at[dstb_v.at[pl.ds(j * C, C)]], rr, sem)

        def wait(j, b):
            rl, rr, sem = bufs[b]
            pltpu.make_async_copy(
                xl_hbm.at[srcb_v.at[pl.ds(j * C, C)]], rl, sem).wait()
            pltpu.make_async_copy(
                xr_hbm.at[dstb_v.at[pl.ds(j * C, C)]], rr, sem).wait()

        def compute(j, b, macc):
            rl, rr, _ = bufs[b]

            @plsc.parallel_loop(0, C, unroll=2)
            def edge_body(i):
                acc = jnp.zeros((16,), jnp.float32)
                for k in range(d // 16):
                    z = rl[i, pl.ds(k * 16, 16)] + rr[i, pl.ds(k * 16, 16)]
                    z = jnp.maximum(z, 0.2 * z)
                    acc = acc + z * att_regs[k]
                acc_v[pl.ds(i * 16, 16)] = acc

            def grp_body(g, macc2):
                fbase = g * 256 + lanes16
                esum = jnp.zeros((16,), jnp.float32)
                for t in range(16):
                    esum = esum + plsc.load_gather(acc_v, [fbase + t])
                e_all[pl.ds(j * C + g * 16, 16)] = esum
                return jnp.maximum(macc2, esum)

            return lax.fori_loop(0, C // 16, grp_body, macc)

        macc = jnp.full((16,), -3.4e38, jnp.float32)
        for h_off, cnt in halves:
            hbase = base + h_off * C
            pltpu.sync_copy(src_hbm.at[pl.ds(hbase, cnt * C)],
                            srcb_v.at[pl.ds(0, cnt * C)])
            pltpu.sync_copy(dst_hbm.at[pl.ds(hbase, cnt * C)],
                            dstb_v.at[pl.ds(0, cnt * C)])
            issue(0, 0)
            p = (cnt - 1) // 2

            def pair(t, m2):
                j = t * 2
                issue(j + 1, 1)
                wait(j, 0)
                m2 = compute(j, 0, m2)
                issue(j + 2, 0)
                wait(j + 1, 1)
                return compute(j + 1, 1, m2)

            macc = lax.fori_loop(0, p, pair, macc)
            if cnt % 2 == 1:
                wait(2 * p, 0)
                macc = compute(2 * p, 0, macc)
            else:
                wait(2 * p, 0)
                issue(2 * p + 1, 1)
                macc = compute(2 * p, 0, macc)
                wait(2 * p + 1, 1)
                macc = compute(2 * p + 1, 1, macc)
            pltpu.sync_copy(e_all.at[pl.ds(0, cnt * C)],
                            e_hbm.at[pl.ds(hbase, cnt * C)])

        mx_v[...] = macc
        pltpu.sync_copy(mx_v, mx_hbm.at[wid])

    return body(xl, xr, src, dst, att)


def _aggregate(xl, src, dstR, dstD, e, mx, n_pad):
    n, d = xl.shape
    e_total = src.shape[0]
    ew = e_total // NW
    nag = ew // C
    PB = 40
    ptail = nag - 3 * PB
    es = e_total // NS
    nd = es // C
    nblk = 25
    ndb = nd // nblk
    rpt = n_pad // NS
    dpt = n_pad // 8
    mesh = plsc.VectorSubcoreMesh(core_axis_name="c", subcore_axis_name="s")

    @functools.partial(
        pl.kernel,
        out_type=[
            jax.ShapeDtypeStruct((NC, n_pad, d), jnp.float32),
            jax.ShapeDtypeStruct((n_pad,), jnp.float32),
        ],
        mesh=mesh,
        compiler_params=pltpu.CompilerParams(needs_layout_passes=False),
        scratch_types=[
            pltpu.VMEM_SHARED((n_pad,), jnp.float32),
            pltpu.VMEM_SHARED((n_pad, d), jnp.float32),
            pltpu.VMEM((PB * C,), jnp.int32),
            pltpu.VMEM((PB, C), jnp.int32),
            pltpu.VMEM((ndb, C), jnp.int32),
            pltpu.VMEM((ndb * C,), jnp.float32),
            pltpu.VMEM((C, d), jnp.float32),
            pltpu.VMEM((C, d), jnp.float32),
            pltpu.VMEM((C, d), jnp.float32),
            pltpu.VMEM((C,), jnp.float32),
            pltpu.VMEM((C,), jnp.float32),
            pltpu.VMEM((C,), jnp.float32),
            pltpu.VMEM((C,), jnp.float32),
            pltpu.VMEM((C,), jnp.float32),
            pltpu.VMEM((C,), jnp.float32),
            pltpu.VMEM((NW, 16), jnp.float32),
            pltpu.VMEM((dpt,), jnp.float32),
            pltpu.VMEM((16,), jnp.float32),
            pltpu.SemaphoreType.DMA,
            pltpu.SemaphoreType.DMA,
            pltpu.SemaphoreType.DMA,
            pltpu.SemaphoreType.DMA,
            pltpu.SemaphoreType.DMA,
            pltpu.SemaphoreType.DMA,
            pltpu.SemaphoreType.DMA,
        ],
    )
    def body(xl_hbm, src_hbm, dstR_hbm, dstD_hbm, e_hbm, mx_hbm,
             outp_hbm, den_hbm,
             den_sh, out_sh, srcb_v, dstb2, dstd2, ed2, r0, r1, r2,
             dc0, dc1, dc2, ec0, ec1, ec2, mx_v, zbuf, red_v,
             gs0, gs1, gs2, ss0, ss1, ss2, dsem):
        c = lax.axis_index("c")
        s = lax.axis_index("s")
        wid = c * NS + s
        zeros16 = jnp.zeros((16,), jnp.float32)
        lanes0 = lax.iota(jnp.int32, 16)
        pltpu.sync_copy(mx_hbm, mx_v)

        @plsc.parallel_loop(0, C, unroll=2)
        def zrow(i):
            for k in range(d // 16):
                r0[i, pl.ds(k * 16, 16)] = zeros16

        @plsc.parallel_loop(0, dpt // 16, unroll=2)
        def zden(i):
            zbuf[pl.ds(i * 16, 16)] = zeros16

        @pl.when(s < 8)
        def _():
            pltpu.sync_copy(zbuf, den_sh.at[pl.ds(s * dpt, dpt)])
        for t in range(rpt // C):
            pltpu.sync_copy(r0, out_sh.at[pl.ds(s * rpt + t * C, C)])
        if rpt % C:
            pltpu.sync_copy(
                r0.at[pl.ds(0, rpt % C)],
                out_sh.at[pl.ds(s * rpt + (rpt // C) * C, rpt % C)])
        plsc.subcore_barrier()

        def mred(i, mv):
            return jnp.maximum(mv, mx_v[i, :])
        mv = lax.fori_loop(0, NW, mred, jnp.full((16,), -3.4e38, jnp.float32))
        for stp in (1, 2, 4, 8):
            red_v[...] = mv
            mv = jnp.maximum(mv, plsc.load_gather(red_v, [lanes0 ^ stp]))
        M = mv

        def dblock(bk, _):
            pltpu.sync_copy(dstD_hbm.at[s, bk], dstd2)
            pltpu.sync_copy(e_hbm.at[pl.ds(s * es + bk * ndb * C, ndb * C)],
                            ed2)

            def dexp(q, _2):
                ed2[pl.ds(q * 16, 16)] = jnp.exp(ed2[pl.ds(q * 16, 16)] - M)
                return 0
            lax.fori_loop(0, ndb * C // 16, dexp, 0)

            def dfire(q, _2):
                pltpu.async_copy(ed2.at[pl.ds(q * C, C)],
                                 den_sh.at[dstd2.at[q]], dsem, add=True)
                return 0
            lax.fori_loop(0, ndb, dfire, 0)
            pltpu.make_async_copy(
                e_hbm.at[pl.ds(s * es, ndb * C)], ed2, dsem).wait()
            return 0
        lax.fori_loop(0, nblk, dblock, 0)
        plsc.subcore_barrier()

        @pl.when(s < 8)
        def _():
            pltpu.sync_copy(den_sh.at[pl.ds(s * dpt, dpt)], zbuf)
            pltpu.sync_copy(zbuf, den_hbm.at[pl.ds(s * dpt, dpt)])
        plsc.subcore_barrier()

        bufs = ((r0, dc0, ec0, gs0, ss0), (r1, dc1, ec1, gs1, ss1),
                (r2, dc2, ec2, gs2, ss2))

        def emit_part(h_off, cnt):
            habs = pl.multiple_of(wid * ew + h_off * C, 8)
            h_off = pl.multiple_of(h_off, 8)

            def gissue(j, b):
                r, dc, ec, gs, _ = bufs[b]
                pltpu.async_copy(xl_hbm.at[srcb_v.at[pl.ds(j * C, C)]],
                                 r, gs)
                pltpu.async_copy(den_hbm.at[dstb2.at[j]], dc, gs)
                pltpu.async_copy(e_hbm.at[pl.ds(habs + j * C, C)], ec, gs)

            def gwait(j, b):
                r, dc, ec, gs, _ = bufs[b]
                pltpu.make_async_copy(
                    xl_hbm.at[srcb_v.at[pl.ds(j * C, C)]], r, gs).wait()
                pltpu.make_async_copy(den_hbm.at[dstb2.at[j]], dc, gs).wait()
                pltpu.make_async_copy(
                    e_hbm.at[pl.ds(habs + j * C, C)], ec, gs).wait()

            def sfire(j, b):
                pltpu.async_copy(bufs[b][0], out_sh.at[dstb2.at[j]],
                                 bufs[b][4], add=True)

            def swait(b):
                pltpu.make_async_copy(bufs[b][0], out_sh.at[dstb2.at[0]],
                                      bufs[b][4]).wait()

            def scale(j, b):
                r, dc, ec, _, _2 = bufs[b]

                def grp(g, _3):
                    den = dc[pl.ds(g * 16, 16)]
                    wv = (jnp.exp(ec[pl.ds(g * 16, 16)] - M)
                          / jnp.maximum(den, 1e-16))
                    ec[pl.ds(g * 16, 16)] = wv
                    return 0
                lax.fori_loop(0, C // 16, grp, 0)

                @plsc.parallel_loop(0, C, unroll=2)
                def sc(i):
                    ws = plsc.load_gather(ec, [jnp.full((16,), i, jnp.int32)])
                    for k in range(d // 16):
                        r[i, pl.ds(k * 16, 16)] = r[i, pl.ds(k * 16, 16)] * ws

            pltpu.sync_copy(src_hbm.at[pl.ds(habs, cnt * C)],
                            srcb_v.at[pl.ds(0, cnt * C)])
            cntp = ((cnt + 7) // 8) * 8
            pltpu.sync_copy(dstR_hbm.at[wid, pl.ds(h_off, cntp)],
                            dstb2.at[pl.ds(0, cntp)])
            gissue(0, 0)
            gwait(0, 0)
            gissue(1, 1)
            scale(0, 0)
            sfire(0, 0)
            gwait(1, 1)
            gissue(2, 2)
            scale(1, 1)
            sfire(1, 1)

            k3 = (cnt - 4) // 3

            def ring(t, _):
                j0 = 2 + t * 3
                for r_ in range(3):
                    j = j0 + r_
                    b = (2 + r_) % 3
                    bn = r_
                    swait(bn)
                    gissue(j + 1, bn)
                    gwait(j, b)
                    scale(j, b)
                    sfire(j, b)
                return 0
            lax.fori_loop(0, k3, ring, 0)

            for j in range(2 + 3 * k3, cnt):
                b = j % 3
                if j + 1 < cnt:
                    bn = (j + 1) % 3
                    swait(bn)
                    gissue(j + 1, bn)
                gwait(j, b)
                scale(j, b)
                sfire(j, b)
            for b in range(3):
                swait(b)

        def part(q, _):
            emit_part(q * PB, PB)
            return 0
        lax.fori_loop(0, 3, part, 0)
        if ptail:
            emit_part(3 * PB, ptail)
        plsc.subcore_barrier()

        pltpu.sync_copy(out_sh.at[pl.ds(s * rpt, rpt)],
                        outp_hbm.at[c, pl.ds(s * rpt, rpt)])

    return body(xl, src, dstR, dstD, e, mx)


def _finalize(a, b, bias2d):
    n_pad, d = a.shape
    fblk = 128

    def body(a_ref, b_ref, bias_ref, o_ref):
        sm = a_ref[...] + b_ref[...] + bias_ref[...]
        neg = jnp.exp(jnp.minimum(sm, 0.0)) - 1.0
        sm = jnp.where(sm > 0, sm, neg)
        z = sm - jnp.max(sm, axis=-1, keepdims=True)
        lse = jnp.log(jnp.sum(jnp.exp(z), axis=-1, keepdims=True))
        o_ref[...] = z - lse

    return pl.pallas_call(
        body,
        grid=(n_pad // fblk,),
        in_specs=[
            pl.BlockSpec((fblk, d), lambda i: (i, 0)),
            pl.BlockSpec((fblk, d), lambda i: (i, 0)),
            pl.BlockSpec((1, d), lambda i: (0, 0)),
        ],
        out_specs=pl.BlockSpec((fblk, d), lambda i: (i, 0)),
        out_shape=jax.ShapeDtypeStruct((n_pad, d), jnp.float32),
    )(a, b, bias2d)


def kernel(x, edge_index, W_l, W_r, att, bias):
    n, d = x.shape
    n_pad = ((n + 127) // 128) * 128
    e_total = edge_index.shape[1]
    ew = e_total // NW
    es = e_total // NS
    src = edge_index[0]
    dst = edge_index[1]
    nag = ew // C
    nagp = ((nag + 7) // 8) * 8
    dstR = jnp.pad(dst.reshape(NW, nag, C),
                   ((0, 0), (0, nagp - nag), (0, 0)))
    dstD = dst.reshape(NS, 25, es // C // 25, C)
    xl, xr = _matmuls(x, W_l, W_r)
    e, mx = _edge_logits(xl, xr, src, dst, att)
    outp, _den = _aggregate(xl, src, dstR, dstD, e, mx, n_pad)
    res = _finalize(outp[0], outp[1], bias.reshape(1, d))
    return res[:n]
